# baseline (speedup 1.0000x reference)
import jax
import jax.numpy as jnp
from jax.experimental import pallas as pl

N = 10000
E = 320000
H = 128
L = 6
B = 64


def _bn(h, g, b):
    m = jnp.mean(h, axis=0)
    v = jnp.var(h, axis=0)
    return g * (h - m) / jnp.sqrt(v + 1e-5) + b


def _final_mlp_kernel(pooled_ref, w1_ref, b1_ref, w2_ref, b2_ref, o_ref):
    r = jnp.maximum(jnp.dot(pooled_ref[...], w1_ref[...],
                            preferred_element_type=jnp.float32) + b1_ref[...], 0.0)
    o_ref[...] = jnp.dot(r, w2_ref[...], preferred_element_type=jnp.float32) + b2_ref[...]


def kernel(x, edge_index, edge_attr, batch, node_emb, edge_emb, lin1_W, lin1_b,
           bn1_g, bn1_b, lin2_W, lin2_b, eps, bn2_g, bn2_b, reg_W1, reg_b1,
           reg_W2, reg_b2):
    h = node_emb[x[:, 0]]
    ea = edge_emb[edge_attr[:, 0]]
    src = edge_index[0]
    dst = edge_index[1]
    for l in range(L):
        msg = jax.nn.relu(h[src] + ea)
        aggr = jax.ops.segment_sum(msg, dst, num_segments=N)
        t = (1.0 + eps[l]) * h + aggr
        t = jax.nn.relu(_bn(t @ lin1_W[l] + lin1_b[l], bn1_g[l], bn1_b[l]))
        t = t @ lin2_W[l] + lin2_b[l]
        h = jax.nn.relu(_bn(h + t, bn2_g[l], bn2_b[l]))
    s = jax.ops.segment_sum(h, batch, num_segments=B)
    cnt = jax.ops.segment_sum(jnp.ones((N,), dtype=h.dtype), batch, num_segments=B)
    mean = s / jnp.maximum(cnt, 1.0)[:, None]
    mx = jax.ops.segment_max(h, batch, num_segments=B)
    mx = jnp.where(cnt[:, None] > 0, mx, 0.0)
    pooled = jnp.concatenate([mean, s, mx], axis=1)
    out2d = pl.pallas_call(
        _final_mlp_kernel,
        out_shape=jax.ShapeDtypeStruct((B, 1), jnp.float32),
    )(pooled, reg_W1, reg_b1[None, :], reg_W2, reg_b2[None, :])
    return out2d[:, 0], h


# speedup vs baseline: 3.4721x; 3.4721x over previous
import functools

import jax
import jax.numpy as jnp
from jax import lax
from jax.experimental import pallas as pl
from jax.experimental.pallas import tpu as pltpu
from jax.experimental.pallas import tpu_sc as plsc

N = 10000
E = 320000
H = 128
L = 6
B = 64

NC = 2
NS = 16
NW = NC * NS
CH = 128
KCH = 80
EPT = KCH * CH
E_PAD = NW * EPT
EROWS = E_PAD // 128

AGG_ROWS = 10240
ZPT = AGG_ROWS // NS


def _prologue_kernel(x_ref, nemb_ref, eemb_ref, srcp_ref, dstp_ref, attrp_ref,
                     h_ref, R_ref, idx_ref, dsto_ref):
    xk = x_ref[...]
    h0 = jnp.zeros((N, H), jnp.float32)
    for k in range(28):
        h0 = h0 + jnp.where(xk == k, 1.0, 0.0) * nemb_ref[k, :]
    h_ref[...] = h0
    for a in range(4):
        R_ref[a] = jnp.maximum(h0 + eemb_ref[a, :], 0.0)
    row = lax.broadcasted_iota(jnp.int32, (EROWS, 128), 0) * 128 \
        + lax.broadcasted_iota(jnp.int32, (EROWS, 128), 1)
    valid = row < E
    idx_ref[...] = jnp.where(valid, attrp_ref[...] * N + srcp_ref[...], 0)
    dsto_ref[...] = jnp.where(valid, dstp_ref[...], N)


def _sc_aggr_body(R_hbm, idx_hbm, dst_hbm, zeros_hbm, out_hbm,
                  idx_v, dst_v, buf, aggr_sh, gsem):
    cid = lax.axis_index("c")
    sid = lax.axis_index("s")
    wid = sid * NC + cid
    pltpu.sync_copy(zeros_hbm, aggr_sh.at[pl.ds(sid * ZPT, ZPT)])
    pltpu.sync_copy(idx_hbm.at[pl.ds(wid * KCH, KCH)], idx_v)
    pltpu.sync_copy(dst_hbm.at[pl.ds(wid * KCH, KCH)], dst_v)
    plsc.subcore_barrier()

    def body(k, carry):
        pltpu.async_copy(R_hbm.at[idx_v.at[k]], buf, gsem).wait()
        pltpu.sync_copy(buf, aggr_sh.at[dst_v.at[k]], add=True)
        return carry

    lax.fori_loop(0, KCH, body, 0)
    plsc.subcore_barrier()

    off = cid * AGG_ROWS + sid * ZPT
    pltpu.sync_copy(aggr_sh.at[pl.ds(sid * ZPT, ZPT)],
                    out_hbm.at[pl.ds(off, ZPT)])


@functools.cache
def _get_sc_aggregate():
    mesh = plsc.VectorSubcoreMesh(
        core_axis_name="c", subcore_axis_name="s",
        num_cores=NC, num_subcores=NS)
    return pl.kernel(
        _sc_aggr_body,
        out_type=jax.ShapeDtypeStruct((NC * AGG_ROWS, H), jnp.float32),
        mesh=mesh,
        scratch_types=[
            pltpu.VMEM((KCH, CH), jnp.int32),
            pltpu.VMEM((KCH, CH), jnp.int32),
            pltpu.VMEM((CH, H), jnp.float32),
            pltpu.VMEM_SHARED((AGG_ROWS, H), jnp.float32),
            pltpu.SemaphoreType.DMA,
        ],
    )


def _sc_aggregate(Rflat, idxp, dstpad, zeros):
    return _get_sc_aggregate()(Rflat, idxp, dstpad, zeros)


def _mm_bf16(a, b):
    return jnp.dot(a.astype(jnp.bfloat16), b.astype(jnp.bfloat16),
                   preferred_element_type=jnp.float32)


def _dense_core(h, aggr_ref, W1_ref, b1_ref, g1_ref, be1_ref,
                W2_ref, b2_ref, g2_ref, be2_ref, eps_ref):
    a = aggr_ref[0:N, :] + aggr_ref[AGG_ROWS:AGG_ROWS + N, :]
    t = (1.0 + eps_ref[0, 0]) * h + a
    u = _mm_bf16(t, W1_ref[...]) + b1_ref[...]
    m = jnp.mean(u, axis=0, keepdims=True)
    v = jnp.mean((u - m) ** 2, axis=0, keepdims=True)
    u = jnp.maximum(g1_ref[...] * (u - m) / jnp.sqrt(v + 1e-5) + be1_ref[...],
                    0.0)
    w = _mm_bf16(u, W2_ref[...]) + b2_ref[...]
    w = h + w
    m2 = jnp.mean(w, axis=0, keepdims=True)
    v2 = jnp.mean((w - m2) ** 2, axis=0, keepdims=True)
    return jnp.maximum(
        g2_ref[...] * (w - m2) / jnp.sqrt(v2 + 1e-5) + be2_ref[...], 0.0)


def _layer_kernel(h_ref, aggr_ref, W1_ref, b1_ref, g1_ref, be1_ref,
                  W2_ref, b2_ref, g2_ref, be2_ref, eps_ref, eemb_ref,
                  ho_ref, R_ref):
    h2 = _dense_core(h_ref[...], aggr_ref, W1_ref, b1_ref, g1_ref, be1_ref,
                     W2_ref, b2_ref, g2_ref, be2_ref, eps_ref)
    ho_ref[...] = h2
    for a in range(4):
        R_ref[a] = jnp.maximum(h2 + eemb_ref[a, :], 0.0)


def _final_kernel(h_ref, aggr_ref, W1_ref, b1_ref, g1_ref, be1_ref,
                  W2_ref, b2_ref, g2_ref, be2_ref, eps_ref, batch_ref,
                  rW1_ref, rb1_ref, rW2_ref, rb2_ref,
                  ho_ref, out_ref, mx_ref):
    h2 = _dense_core(h_ref[...], aggr_ref, W1_ref, b1_ref, g1_ref, be1_ref,
                     W2_ref, b2_ref, g2_ref, be2_ref, eps_ref)
    ho_ref[...] = h2
    bvec = batch_ref[...]
    oh = (bvec == lax.broadcasted_iota(jnp.int32, (N, B), 1)
          ).astype(jnp.float32)
    s = lax.dot_general(oh, h2, dimension_numbers=(((0,), (0,)), ((), ())),
                        preferred_element_type=jnp.float32, precision=lax.Precision.HIGHEST)
    cnt = jnp.sum(oh, axis=0)
    mean = s / jnp.maximum(cnt, 1.0)[:, None]

    def body(b, carry):
        mxb = jnp.max(jnp.where(bvec == b, h2, 0.0), axis=0, keepdims=True)
        mx_ref[pl.ds(b, 1), :] = mxb
        return carry

    lax.fori_loop(0, B, body, 0)
    mx = mx_ref[...]

    r = jnp.maximum(
        _mm_bf16(mean, rW1_ref[0:H, :])
        + _mm_bf16(s, rW1_ref[H:2 * H, :])
        + _mm_bf16(mx, rW1_ref[2 * H:3 * H, :])
        + rb1_ref[...], 0.0)
    out_ref[...] = _mm_bf16(r, rW2_ref[...]) + rb2_ref[...]


def kernel(x, edge_index, edge_attr, batch, node_emb, edge_emb, lin1_W,
           lin1_b, bn1_g, bn1_b, lin2_W, lin2_b, eps, bn2_g, bn2_b,
           reg_W1, reg_b1, reg_W2, reg_b2):
    x_ = x.astype(jnp.int32).reshape(N, 1)
    srcp = jnp.pad(edge_index[0].astype(jnp.int32), (0, E_PAD - E)
                   ).reshape(EROWS, 128)
    dstp = jnp.pad(edge_index[1].astype(jnp.int32), (0, E_PAD - E)
                   ).reshape(EROWS, 128)
    attrp = jnp.pad(edge_attr[:, 0].astype(jnp.int32), (0, E_PAD - E)
                    ).reshape(EROWS, 128)
    zeros = jnp.zeros((ZPT, H), jnp.float32)
    batch2d = batch.astype(jnp.int32).reshape(N, 1)

    f32 = jnp.float32
    h, R, idxp, dstpad = pl.pallas_call(
        _prologue_kernel,
        out_shape=[
            jax.ShapeDtypeStruct((N, H), f32),
            jax.ShapeDtypeStruct((4, N, H), f32),
            jax.ShapeDtypeStruct((EROWS, 128), jnp.int32),
            jax.ShapeDtypeStruct((EROWS, 128), jnp.int32),
        ],
    )(x_, node_emb, edge_emb, srcp, dstp, attrp)

    out2d = None
    for l in range(L):
        aggr2 = _sc_aggregate(R.reshape(4 * N, H), idxp, dstpad, zeros)

        wargs = (lin1_W[l], lin1_b[l][None, :], bn1_g[l][None, :],
                 bn1_b[l][None, :], lin2_W[l], lin2_b[l][None, :],
                 bn2_g[l][None, :], bn2_b[l][None, :], eps[l].reshape(1, 1))
        if l < L - 1:
            h, R = pl.pallas_call(
                _layer_kernel,
                out_shape=[
                    jax.ShapeDtypeStruct((N, H), f32),
                    jax.ShapeDtypeStruct((4, N, H), f32),
                ],
            )(h, aggr2, *wargs, edge_emb)
        else:
            h, out2d = pl.pallas_call(
                _final_kernel,
                out_shape=[
                    jax.ShapeDtypeStruct((N, H), f32),
                    jax.ShapeDtypeStruct((B, 1), f32),
                ],
                scratch_shapes=[pltpu.VMEM((B, H), f32)],
            )(h, aggr2, *wargs, batch2d, reg_W1, reg_b1[None, :],
              reg_W2, reg_b2[None, :])
    return out2d[:, 0], h


# speedup vs baseline: 3.9976x; 1.1513x over previous
import functools

import jax
import jax.numpy as jnp
from jax import lax
from jax.experimental import pallas as pl
from jax.experimental.pallas import tpu as pltpu
from jax.experimental.pallas import tpu_sc as plsc

N = 10000
E = 320000
H = 128
L = 6
B = 64

NC = 2
NS = 16
NW = NC * NS
CH = 128
KCH = 80
EPT = KCH * CH
E_PAD = NW * EPT
EROWS = E_PAD // 128

AGG_ROWS = 10240
ZPT = AGG_ROWS // NS


def _prologue_kernel(x_ref, nemb_ref, eemb_ref, srcp_ref, dstp_ref, attrp_ref,
                     h_ref, R_ref, idx_ref, dsto_ref):
    xk = x_ref[...]
    h0 = jnp.zeros((N, H), jnp.float32)
    for k in range(28):
        h0 = h0 + jnp.where(xk == k, 1.0, 0.0) * nemb_ref[k, :]
    h_ref[...] = h0
    for a in range(4):
        R_ref[a] = jnp.maximum(h0 + eemb_ref[a, :], 0.0)
    row = lax.broadcasted_iota(jnp.int32, (EROWS, 128), 0) * 128 \
        + lax.broadcasted_iota(jnp.int32, (EROWS, 128), 1)
    valid = row < E
    idx_ref[...] = jnp.where(valid, attrp_ref[...] * N + srcp_ref[...], 0)
    dsto_ref[...] = jnp.where(valid, dstp_ref[...], N)


def _sc_aggr_body(R_hbm, idx_hbm, dst_hbm, zeros_hbm, out_hbm,
                  idx_v, dst_v, bufA, bufB, aggr_sh, semA, semB):
    cid = lax.axis_index("c")
    sid = lax.axis_index("s")
    wid = sid * NC + cid
    pltpu.sync_copy(zeros_hbm, aggr_sh.at[pl.ds(sid * ZPT, ZPT)])
    plsc.subcore_barrier()

    def gather(k, buf, sem):
        return pltpu.async_copy(R_hbm.at[idx_v.at[k]], buf, sem)

    def wait(k, buf, sem):
        pltpu.make_async_copy(R_hbm.at[idx_v.at[k]], buf, sem).wait()

    def body(k2, carry):
        k = 2 * k2
        gather(k + 1, bufB, semB)
        wait(k, bufA, semA)
        pltpu.sync_copy(bufA, aggr_sh.at[dst_v.at[k]], add=True)
        gather(k + 2, bufA, semA)
        wait(k + 1, bufB, semB)
        pltpu.sync_copy(bufB, aggr_sh.at[dst_v.at[k + 1]], add=True)
        return carry

    HK = KCH // 2
    for half in range(2):
        base = wid * KCH + half * HK
        pltpu.sync_copy(idx_hbm.at[pl.ds(base, HK)], idx_v)
        pltpu.sync_copy(dst_hbm.at[pl.ds(base, HK)], dst_v)
        gather(0, bufA, semA)
        lax.fori_loop(0, HK // 2 - 1, body, 0)
        gather(HK - 1, bufB, semB)
        wait(HK - 2, bufA, semA)
        pltpu.sync_copy(bufA, aggr_sh.at[dst_v.at[HK - 2]], add=True)
        wait(HK - 1, bufB, semB)
        pltpu.sync_copy(bufB, aggr_sh.at[dst_v.at[HK - 1]], add=True)
    plsc.subcore_barrier()

    off = cid * AGG_ROWS + sid * ZPT
    pltpu.sync_copy(aggr_sh.at[pl.ds(sid * ZPT, ZPT)],
                    out_hbm.at[pl.ds(off, ZPT)])


@functools.cache
def _get_sc_aggregate():
    mesh = plsc.VectorSubcoreMesh(
        core_axis_name="c", subcore_axis_name="s",
        num_cores=NC, num_subcores=NS)
    return pl.kernel(
        _sc_aggr_body,
        out_type=jax.ShapeDtypeStruct((NC * AGG_ROWS, H), jnp.float32),
        mesh=mesh,
        scratch_types=[
            pltpu.VMEM((KCH // 2, CH), jnp.int32),
            pltpu.VMEM((KCH // 2, CH), jnp.int32),
            pltpu.VMEM((CH, H), jnp.float32),
            pltpu.VMEM((CH, H), jnp.float32),
            pltpu.VMEM_SHARED((AGG_ROWS, H), jnp.float32),
            pltpu.SemaphoreType.DMA,
            pltpu.SemaphoreType.DMA,
        ],
    )


def _sc_aggregate(Rflat, idxp, dstpad, zeros):
    return _get_sc_aggregate()(Rflat, idxp, dstpad, zeros)


def _mm_bf16(a, b):
    return jnp.dot(a.astype(jnp.bfloat16), b.astype(jnp.bfloat16),
                   preferred_element_type=jnp.float32)


def _dense_core(h, aggr_ref, W1_ref, b1_ref, g1_ref, be1_ref,
                W2_ref, b2_ref, g2_ref, be2_ref, eps_ref):
    a = aggr_ref[0:N, :] + aggr_ref[AGG_ROWS:AGG_ROWS + N, :]
    t = (1.0 + eps_ref[0, 0]) * h + a
    u = _mm_bf16(t, W1_ref[...]) + b1_ref[...]
    m = jnp.mean(u, axis=0, keepdims=True)
    v = jnp.mean((u - m) ** 2, axis=0, keepdims=True)
    u = jnp.maximum(g1_ref[...] * (u - m) / jnp.sqrt(v + 1e-5) + be1_ref[...],
                    0.0)
    w = _mm_bf16(u, W2_ref[...]) + b2_ref[...]
    w = h + w
    m2 = jnp.mean(w, axis=0, keepdims=True)
    v2 = jnp.mean((w - m2) ** 2, axis=0, keepdims=True)
    return jnp.maximum(
        g2_ref[...] * (w - m2) / jnp.sqrt(v2 + 1e-5) + be2_ref[...], 0.0)


def _layer_kernel(h_ref, aggr_ref, W1_ref, b1_ref, g1_ref, be1_ref,
                  W2_ref, b2_ref, g2_ref, be2_ref, eps_ref, eemb_ref,
                  ho_ref, R_ref):
    h2 = _dense_core(h_ref[...], aggr_ref, W1_ref, b1_ref, g1_ref, be1_ref,
                     W2_ref, b2_ref, g2_ref, be2_ref, eps_ref)
    ho_ref[...] = h2
    for a in range(4):
        R_ref[a] = jnp.maximum(h2 + eemb_ref[a, :], 0.0)


def _final_kernel(h_ref, aggr_ref, W1_ref, b1_ref, g1_ref, be1_ref,
                  W2_ref, b2_ref, g2_ref, be2_ref, eps_ref, batch_ref,
                  rW1_ref, rb1_ref, rW2_ref, rb2_ref,
                  ho_ref, out_ref, mx_ref):
    h2 = _dense_core(h_ref[...], aggr_ref, W1_ref, b1_ref, g1_ref, be1_ref,
                     W2_ref, b2_ref, g2_ref, be2_ref, eps_ref)
    ho_ref[...] = h2
    bvec = batch_ref[...]
    oh = (bvec == lax.broadcasted_iota(jnp.int32, (N, B), 1)
          ).astype(jnp.float32)
    s = lax.dot_general(oh, h2, dimension_numbers=(((0,), (0,)), ((), ())),
                        preferred_element_type=jnp.float32, precision=lax.Precision.HIGHEST)
    cnt = jnp.sum(oh, axis=0)
    mean = s / jnp.maximum(cnt, 1.0)[:, None]

    def body(b, carry):
        mxb = jnp.max(jnp.where(bvec == b, h2, 0.0), axis=0, keepdims=True)
        mx_ref[pl.ds(b, 1), :] = mxb
        return carry

    lax.fori_loop(0, B, body, 0)
    mx = mx_ref[...]

    r = jnp.maximum(
        _mm_bf16(mean, rW1_ref[0:H, :])
        + _mm_bf16(s, rW1_ref[H:2 * H, :])
        + _mm_bf16(mx, rW1_ref[2 * H:3 * H, :])
        + rb1_ref[...], 0.0)
    out_ref[...] = _mm_bf16(r, rW2_ref[...]) + rb2_ref[...]


def kernel(x, edge_index, edge_attr, batch, node_emb, edge_emb, lin1_W,
           lin1_b, bn1_g, bn1_b, lin2_W, lin2_b, eps, bn2_g, bn2_b,
           reg_W1, reg_b1, reg_W2, reg_b2):
    x_ = x.astype(jnp.int32).reshape(N, 1)
    srcp = jnp.pad(edge_index[0].astype(jnp.int32), (0, E_PAD - E)
                   ).reshape(EROWS, 128)
    dstp = jnp.pad(edge_index[1].astype(jnp.int32), (0, E_PAD - E)
                   ).reshape(EROWS, 128)
    attrp = jnp.pad(edge_attr[:, 0].astype(jnp.int32), (0, E_PAD - E)
                    ).reshape(EROWS, 128)
    zeros = jnp.zeros((ZPT, H), jnp.float32)
    batch2d = batch.astype(jnp.int32).reshape(N, 1)

    f32 = jnp.float32
    h, R, idxp, dstpad = pl.pallas_call(
        _prologue_kernel,
        out_shape=[
            jax.ShapeDtypeStruct((N, H), f32),
            jax.ShapeDtypeStruct((4, N, H), f32),
            jax.ShapeDtypeStruct((EROWS, 128), jnp.int32),
            jax.ShapeDtypeStruct((EROWS, 128), jnp.int32),
        ],
    )(x_, node_emb, edge_emb, srcp, dstp, attrp)

    out2d = None
    for l in range(L):
        aggr2 = _sc_aggregate(R.reshape(4 * N, H), idxp, dstpad, zeros)

        wargs = (lin1_W[l], lin1_b[l][None, :], bn1_g[l][None, :],
                 bn1_b[l][None, :], lin2_W[l], lin2_b[l][None, :],
                 bn2_g[l][None, :], bn2_b[l][None, :], eps[l].reshape(1, 1))
        if l < L - 1:
            h, R = pl.pallas_call(
                _layer_kernel,
                out_shape=[
                    jax.ShapeDtypeStruct((N, H), f32),
                    jax.ShapeDtypeStruct((4, N, H), f32),
                ],
            )(h, aggr2, *wargs, edge_emb)
        else:
            h, out2d = pl.pallas_call(
                _final_kernel,
                out_shape=[
                    jax.ShapeDtypeStruct((N, H), f32),
                    jax.ShapeDtypeStruct((B, 1), f32),
                ],
                scratch_shapes=[pltpu.VMEM((B, H), f32)],
            )(h, aggr2, *wargs, batch2d, reg_W1, reg_b1[None, :],
              reg_W2, reg_b2[None, :])
    return out2d[:, 0], h
